# baseline (speedup 1.0000x reference)
import jax
import jax.numpy as jnp
from jax import lax
from jax.experimental import pallas as pl
from jax.experimental.pallas import tpu as pltpu
from jax.experimental.pallas import tpu_sc as plsc

N = 10000
D = 256
NC, NS = 2, 16
H = D // NC
NP = 2
NH = 5120
NPAD = NP * NH
AROWS = 5248
RZ = AROWS // NS
RW = NH // NS
CH = 128
K = 80
NB = 2
EPAD = NS * K * CH


def _fill(ref, val):
    v = jnp.full((16,), val, jnp.float32)

    def _f(i, carry):
        r = i // (H // 16)
        c = i % (H // 16)
        ref[r, pl.ds(c * 16, 16)] = v
        return carry

    lax.fori_loop(0, 128 * (H // 16), _f, 0)


def _zero_slice(zbuf, shared, wid):
    for off, nr in ((0, 128), (128, 128), (256, RZ - 256)):
        pltpu.sync_copy(zbuf.at[pl.ds(0, nr)],
                        shared.at[pl.ds(wid * RZ + off, nr)])


EPT = K * CH


def _sc_scatter_body(h_hbm, src_hbm, dst_hbm, cnt_hbm, out_hbm,
                     src_v, dst_v, cv, g, zbuf, shared, g0, g1, s0, s1):
    gsems = (g0, g1)
    ssems = (s0, s1)
    cid = lax.axis_index("c")
    wid = lax.axis_index("s")
    _fill(zbuf, 0.0)

    hview = h_hbm.at[cid]
    for p in range(NP):
        pltpu.sync_copy(src_hbm.at[p, wid], src_v)
        pltpu.sync_copy(dst_hbm.at[p, wid], dst_v)
        pltpu.sync_copy(cnt_hbm.at[p, wid], cv)
        _zero_slice(zbuf, shared, wid)
        plsc.subcore_barrier()
        e = cv[pl.ds(0, 16)][0]
        nit = (e + (NB * CH - 1)) // (NB * CH)

        def _chunk(i, carry):
            j = i * NB
            cps = []
            for b in range(NB):
                @pl.when(i > 0)
                def _():
                    pltpu.make_async_copy(
                        g.at[b], shared.at[dst_v.at[j + b]], ssems[b]).wait()
                cps.append(pltpu.async_copy(hview.at[src_v.at[j + b]],
                                            g.at[b], gsems[b]))
            for b in range(NB):
                cps[b].wait()
                pltpu.async_copy(g.at[b], shared.at[dst_v.at[j + b]],
                                 ssems[b], add=True)
            return carry

        lax.fori_loop(0, nit, _chunk, 0)
        for b in range(NB):
            @pl.when(nit > 0)
            def _():
                pltpu.make_async_copy(
                    g.at[b], shared.at[dst_v.at[b]], ssems[b]).wait()
        plsc.subcore_barrier()
        pltpu.sync_copy(shared.at[pl.ds(wid * RW, RW)],
                        out_hbm.at[cid, pl.ds(p * NH + wid * RW, RW)])


_scatter_call = pl.kernel(
    _sc_scatter_body,
    out_type=jax.ShapeDtypeStruct((NC, NPAD, H), jnp.float32),
    mesh=plsc.VectorSubcoreMesh(core_axis_name="c", subcore_axis_name="s"),
    scratch_types=[
        pltpu.VMEM((K, CH), jnp.int32),
        pltpu.VMEM((K, CH), jnp.int32),
        pltpu.VMEM((16,), jnp.int32),
        pltpu.VMEM((NB, CH, H), jnp.float32),
        pltpu.VMEM((128, H), jnp.float32),
        pltpu.VMEM_SHARED((AROWS, H), jnp.float32),
        pltpu.SemaphoreType.DMA,
        pltpu.SemaphoreType.DMA,
        pltpu.SemaphoreType.DMA,
        pltpu.SemaphoreType.DMA,
    ],
)


def _tc_layer_body(s_ref, h_ref, c_ref, w_ref, b_ref, o_ref):
    cnt = c_ref[...] + 1.0
    inv = 1.0 / cnt
    a = jnp.concatenate(
        [(s_ref[c] + h_ref[c]) * inv for c in range(NC)], axis=1)
    y = lax.dot_general(a, w_ref[...], (((1,), (1,)), ((), ())),
                        preferred_element_type=jnp.float32)
    y = jnp.maximum(y + b_ref[...], 0.0)
    for c in range(NC):
        o_ref[c] = y[:, c * H:(c + 1) * H]


_BN = 1000


def _tc_layer(s, h, c2, w, b2):
    return pl.pallas_call(
        _tc_layer_body,
        grid=(N // _BN,),
        in_specs=[
            pl.BlockSpec((NC, _BN, H), lambda i: (0, i, 0)),
            pl.BlockSpec((NC, _BN, H), lambda i: (0, i, 0)),
            pl.BlockSpec((_BN, 1), lambda i: (i, 0)),
            pl.BlockSpec((D, D), lambda i: (0, 0)),
            pl.BlockSpec((1, D), lambda i: (0, 0)),
        ],
        out_specs=pl.BlockSpec((NC, _BN, H), lambda i: (0, i, 0)),
        out_shape=jax.ShapeDtypeStruct((NC, N, H), jnp.float32),
    )(s, h, c2, w, b2)


@jax.jit
def _impl(x, edge_index, W, b):
    src = edge_index[0]
    dst = edge_index[1]
    e = src.shape[0]
    m = dst < NH
    skey = (dst >= NH).astype(jnp.int32)
    n0 = e - jnp.sum(skey)
    packed = (src << 13) | jnp.where(m, dst, dst - NH)
    _, packed_s = lax.sort((skey, packed), num_keys=1, is_stable=False)
    t = jnp.arange(EPAD, dtype=jnp.int32)
    v0 = jnp.take(packed_s, jnp.minimum(t, e - 1))
    f0 = jnp.where(t < n0, v0, NH)
    i1 = n0 + t
    v1 = jnp.take(packed_s, jnp.minimum(i1, e - 1))
    f1 = jnp.where(i1 < e, v1, NH)
    flat = jnp.concatenate([f0, f1])
    osrc = (flat >> 13).reshape(NP, K, NS, CH).transpose(0, 2, 1, 3)
    odst = (flat & (8192 - 1)).reshape(NP, K, NS, CH).transpose(0, 2, 1, 3)
    n = jnp.stack([n0, e - n0])
    slot = (jnp.arange(K)[:, None] * NS + jnp.arange(NS)[None, :]) * CH
    cap = jnp.clip(n[:, None, None] - slot[None], 0, CH)
    ecnt = cap.sum(axis=1).astype(jnp.int32)
    ocnt = jnp.broadcast_to(ecnt[:, :, None], (NP, NS, 16)).astype(jnp.int32)
    ones_h = jnp.ones((NC, N, H), jnp.float32)
    c2 = _scatter_call(ones_h, osrc, odst, ocnt)[0, :N, 0:1]
    h = jnp.stack([x[:, :H], x[:, H:]])
    for i in range(W.shape[0]):
        s = _scatter_call(h, osrc, odst, ocnt)
        h = _tc_layer(s, h, c2, W[i], b[i].reshape(1, D))
    return jnp.concatenate([h[0], h[1]], axis=1)


def kernel(x, edge_index, W, b):
    return _impl(x, edge_index, W, b)


# speedup vs baseline: 1.0048x; 1.0048x over previous
import jax
import jax.numpy as jnp
from jax import lax
from jax.experimental import pallas as pl
from jax.experimental.pallas import tpu as pltpu
from jax.experimental.pallas import tpu_sc as plsc

N = 10000
D = 256
NC, NS = 2, 16
H = D // NC
NP = 2
NH = 5120
NPAD = NP * NH
AROWS = 5248
RZ = AROWS // NS
RW = NH // NS
CH = 128
K = 80
NB = 2
EPAD = NS * K * CH


def _fill(ref, val):
    v = jnp.full((16,), val, jnp.float32)

    def _f(i, carry):
        r = i // (H // 16)
        c = i % (H // 16)
        ref[r, pl.ds(c * 16, 16)] = v
        return carry

    lax.fori_loop(0, 128 * (H // 16), _f, 0)


def _zero_slice(zbuf, shared, wid):
    for off, nr in ((0, 128), (128, 128), (256, RZ - 256)):
        pltpu.sync_copy(zbuf.at[pl.ds(0, nr)],
                        shared.at[pl.ds(wid * RZ + off, nr)])


EPT = K * CH


def _sc_scatter_body(h_hbm, src_hbm, dst_hbm, cnt_hbm, out_hbm,
                     src_v, dst_v, cv, g, zbuf, shared, g0, g1, s0, s1):
    gsems = (g0, g1)
    ssems = (s0, s1)
    cid = lax.axis_index("c")
    wid = lax.axis_index("s")
    _fill(zbuf, 0.0)

    hview = h_hbm.at[cid]
    for p in range(NP):
        pltpu.sync_copy(src_hbm.at[p, wid], src_v)
        pltpu.sync_copy(dst_hbm.at[p, wid], dst_v)
        pltpu.sync_copy(cnt_hbm.at[p, wid], cv)
        _zero_slice(zbuf, shared, wid)
        plsc.subcore_barrier()
        e = cv[pl.ds(0, 16)][0]
        nit = (e + (NB * CH - 1)) // (NB * CH)

        def _chunk(i, carry):
            j = i * NB
            cps = []
            for b in range(NB):
                @pl.when(i > 0)
                def _():
                    pltpu.make_async_copy(
                        g.at[b], shared.at[dst_v.at[j + b]], ssems[b]).wait()
                cps.append(pltpu.async_copy(hview.at[src_v.at[j + b]],
                                            g.at[b], gsems[b]))
            for b in range(NB):
                cps[b].wait()
                pltpu.async_copy(g.at[b], shared.at[dst_v.at[j + b]],
                                 ssems[b], add=True)
            return carry

        lax.fori_loop(0, nit, _chunk, 0)
        for b in range(NB):
            @pl.when(nit > 0)
            def _():
                pltpu.make_async_copy(
                    g.at[b], shared.at[dst_v.at[b]], ssems[b]).wait()
        plsc.subcore_barrier()
        pltpu.sync_copy(shared.at[pl.ds(wid * RW, RW)],
                        out_hbm.at[cid, pl.ds(p * NH + wid * RW, RW)])


_scatter_call = pl.kernel(
    _sc_scatter_body,
    out_type=jax.ShapeDtypeStruct((NC, NPAD, H), jnp.float32),
    mesh=plsc.VectorSubcoreMesh(core_axis_name="c", subcore_axis_name="s"),
    scratch_types=[
        pltpu.VMEM((K, CH), jnp.int32),
        pltpu.VMEM((K, CH), jnp.int32),
        pltpu.VMEM((16,), jnp.int32),
        pltpu.VMEM((NB, CH, H), jnp.float32),
        pltpu.VMEM((128, H), jnp.float32),
        pltpu.VMEM_SHARED((AROWS, H), jnp.float32),
        pltpu.SemaphoreType.DMA,
        pltpu.SemaphoreType.DMA,
        pltpu.SemaphoreType.DMA,
        pltpu.SemaphoreType.DMA,
    ],
)


def _tc_layer_body(s_ref, h_ref, c_ref, w_ref, b_ref, o_ref):
    cnt = c_ref[...] + 1.0
    inv = 1.0 / cnt
    a = jnp.concatenate(
        [(s_ref[c] + h_ref[c]) * inv for c in range(NC)], axis=1)
    y = lax.dot_general(a, w_ref[...], (((1,), (1,)), ((), ())),
                        preferred_element_type=jnp.float32)
    y = jnp.maximum(y + b_ref[...], 0.0)
    for c in range(NC):
        o_ref[c] = y[:, c * H:(c + 1) * H]


_BN = 1000


def _tc_layer(s, h, c2, w, b2):
    return pl.pallas_call(
        _tc_layer_body,
        grid=(N // _BN,),
        in_specs=[
            pl.BlockSpec((NC, _BN, H), lambda i: (0, i, 0)),
            pl.BlockSpec((NC, _BN, H), lambda i: (0, i, 0)),
            pl.BlockSpec((_BN, 1), lambda i: (i, 0)),
            pl.BlockSpec((D, D), lambda i: (0, 0)),
            pl.BlockSpec((1, D), lambda i: (0, 0)),
        ],
        out_specs=pl.BlockSpec((NC, _BN, H), lambda i: (0, i, 0)),
        out_shape=jax.ShapeDtypeStruct((NC, N, H), jnp.float32),
    )(s, h, c2, w, b2)


@jax.jit
def _impl(x, edge_index, W, b):
    src = edge_index[0]
    dst = edge_index[1]
    e = src.shape[0]
    m = dst < NH
    mi = m.astype(jnp.int32)
    c0 = jnp.cumsum(mi)
    n0 = c0[e - 1]
    c1 = (1 + jnp.arange(e, dtype=jnp.int32)) - c0
    pos = jnp.where(m, c0, c1) - 1
    idx = (1 - mi) * EPAD + pos
    packed = (src << 13) | jnp.where(m, dst, dst - NH)
    flat = jnp.full((NP * EPAD,), NH, jnp.int32)
    flat = flat.at[idx].set(packed, mode='drop', unique_indices=True)
    osrc = (flat >> 13).reshape(NP, K, NS, CH).transpose(0, 2, 1, 3)
    odst = (flat & (8192 - 1)).reshape(NP, K, NS, CH).transpose(0, 2, 1, 3)
    n = jnp.stack([n0, e - n0])
    slot = (jnp.arange(K)[:, None] * NS + jnp.arange(NS)[None, :]) * CH
    cap = jnp.clip(n[:, None, None] - slot[None], 0, CH)
    ecnt = cap.sum(axis=1).astype(jnp.int32)
    ocnt = jnp.broadcast_to(ecnt[:, :, None], (NP, NS, 16)).astype(jnp.int32)
    ones_h = jnp.ones((NC, N, H), jnp.float32)
    c2 = _scatter_call(ones_h, osrc, odst, ocnt)[0, :N, 0:1]
    h = jnp.stack([x[:, :H], x[:, H:]])
    for i in range(W.shape[0]):
        s = _scatter_call(h, osrc, odst, ocnt)
        h = _tc_layer(s, h, c2, W[i], b[i].reshape(1, D))
    return jnp.concatenate([h[0], h[1]], axis=1)


def kernel(x, edge_index, W, b):
    return _impl(x, edge_index, W, b)
